# baseline (speedup 1.0000x reference)
import functools

import jax
import jax.numpy as jnp
from jax import lax
from jax.experimental import pallas as pl
from jax.experimental.pallas import tpu as pltpu
from jax.experimental.pallas import tpu_sc as plsc

N = 10000
E = 320000
D = 128
H = 256
L = 3

NC = 2
NS = 16
NWORK = NC * NS
CHUNK = 128
NCHUNKS = E // CHUNK
ITERS = (NCHUNKS + NWORK - 1) // NWORK
OUTER = (ITERS + 1) // 2
RPS = 624


def _sc_body(do_gather, e_hbm, src_hbm, dst_hbm, h_hbm, out_hbm,
             src0, src1, dst0, dst1, ev0, ev1, rows0, rows1, agg_sh,
             sl0, sl1, sd0, sd1, se0, se1, sg0, sg1):
    c = lax.axis_index("c")
    s = lax.axis_index("s")
    wid = s * NC + c
    sets = ((src0, dst0, ev0, rows0, sl0, sd0, se0, sg0),
            (src1, dst1, ev1, rows1, sl1, sd1, se1, sg1))

    def _zrow(r, carry):
        for j in range(8):
            rows0[r, pl.ds(j * 16, 16)] = jnp.zeros((16,), jnp.float32)
        return carry
    lax.fori_loop(0, CHUNK, _zrow, 0)
    for off, n in ((0, 128), (128, 128), (256, 128), (384, 128), (512, 112)):
        pltpu.sync_copy(rows0.at[pl.ds(0, n)],
                        agg_sh.at[pl.ds(s * RPS + off, n)])

    @pl.when(s == NS - 1)
    def _ztail():
        pltpu.sync_copy(rows0.at[pl.ds(0, 16)], agg_sh.at[pl.ds(NS * RPS, 16)])
    plsc.subcore_barrier()

    def _lin(cid, bufs, issue):
        srcb, dstb, evb = bufs[0], bufs[1], bufs[2]
        slb, sdb, seb = bufs[4], bufs[5], bufs[6]
        for hbm, vb, sem in ((src_hbm, srcb, slb), (dst_hbm, dstb, sdb)):
            cp = pltpu.make_async_copy(hbm.at[pl.ds(cid, 1)], vb, sem)
            cp.start() if issue else cp.wait()
        cp = pltpu.make_async_copy(
            e_hbm.at[pl.ds(cid * (CHUNK // 2), CHUNK // 2)], evb, seb)
        cp.start() if issue else cp.wait()

    def _gat(bufs, issue):
        cp = pltpu.make_async_copy(h_hbm.at[bufs[0].at[0]], bufs[3], bufs[7])
        cp.start() if issue else cp.wait()

    _lin(wid, sets[0], True)
    _lin(wid + NWORK, sets[1], True)
    if do_gather:
        _lin(wid, sets[0], False)
        _gat(sets[0], True)

    def _outer(o, carry):
        for b in range(2):
            cid = wid + (2 * o + b) * NWORK
            bufs = sets[b]
            nbufs = sets[1 - b]
            dstb, evb, rowsb = bufs[1], bufs[2], bufs[3]

            @pl.when(cid < NCHUNKS)
            def _():
                if do_gather:
                    @pl.when(cid + NWORK < NCHUNKS)
                    def _():
                        _lin(cid + NWORK, nbufs, False)
                        _gat(nbufs, True)
                    _gat(bufs, False)
                else:
                    _lin(cid, bufs, False)

                def _row(rp, carry2):
                    for j in range(8):
                        sl = pl.ds(j * 16, 16)
                        w = evb[rp, sl]
                        lo = lax.bitcast_convert_type(w << 16, jnp.float32)
                        hi = lax.bitcast_convert_type(
                            w & jnp.int32(-65536), jnp.float32)
                        for half, part in ((0, lo), (1, hi)):
                            m = part
                            if do_gather:
                                m = m + rowsb[2 * rp + half, sl]
                            rowsb[2 * rp + half, sl] = jnp.maximum(m, 0.0)
                    return carry2
                lax.fori_loop(0, CHUNK // 2, _row, 0)

                pltpu.sync_copy(rowsb, agg_sh.at[dstb.at[0]], add=True)

                @pl.when(cid + 2 * NWORK < NCHUNKS)
                def _():
                    _lin(cid + 2 * NWORK, bufs, True)
        return carry
    lax.fori_loop(0, OUTER, _outer, 0)
    plsc.subcore_barrier()

    for off, n in ((0, 128), (128, 128), (256, 128), (384, 128), (512, 112)):
        pltpu.sync_copy(agg_sh.at[pl.ds(s * RPS + off, n)],
                        out_hbm.at[c, pl.ds(s * RPS + off, n)])

    @pl.when(s == NS - 1)
    def _wtail():
        pltpu.sync_copy(agg_sh.at[pl.ds(NS * RPS, 16)],
                        out_hbm.at[c, pl.ds(NS * RPS, 16)])


def _make_sc(do_gather):
    mesh = plsc.VectorSubcoreMesh(core_axis_name="c", subcore_axis_name="s")
    return pl.kernel(
        functools.partial(_sc_body, do_gather),
        out_type=jax.ShapeDtypeStruct((NC, N, D), jnp.float32),
        mesh=mesh,
        scratch_types=[
            pltpu.VMEM((1, 128), jnp.int32),
            pltpu.VMEM((1, 128), jnp.int32),
            pltpu.VMEM((1, 128), jnp.int32),
            pltpu.VMEM((1, 128), jnp.int32),
            pltpu.VMEM((CHUNK // 2, D), jnp.int32),
            pltpu.VMEM((CHUNK // 2, D), jnp.int32),
            pltpu.VMEM((CHUNK, D), jnp.float32),
            pltpu.VMEM((CHUNK, D), jnp.float32),
            pltpu.VMEM_SHARED((N, D), jnp.float32),
        ] + [pltpu.SemaphoreType.DMA] * 8,
    )


def _enc_body(ea_ref, we_ref, be_ref, *out_refs):
    ea = ea_ref[...]
    for l, out_ref in enumerate(out_refs):
        r = (jnp.dot(ea, we_ref[l], preferred_element_type=jnp.float32)
             + be_ref[l])
        rp = r.reshape(r.shape[0] // 2, 2, D)
        lo = jax.lax.bitcast_convert_type(
            rp[:, 0, :].astype(jnp.bfloat16), jnp.uint16).astype(jnp.uint32)
        hi = jax.lax.bitcast_convert_type(
            rp[:, 1, :].astype(jnp.bfloat16), jnp.uint16).astype(jnp.uint32)
        out_ref[...] = jax.lax.bitcast_convert_type(
            lo | (hi << 16), jnp.int32)


def _mlp_body(last, h_ref, agg_ref, ep_ref, w1_ref, b1_ref, w2_ref, b2_ref, out_ref):
    t = ep_ref[...] * h_ref[...] + agg_ref[0] + agg_ref[1]
    t = jnp.maximum(jnp.dot(t, w1_ref[...], preferred_element_type=jnp.float32)
                    + b1_ref[...], 0.0)
    o = jnp.dot(t, w2_ref[...], preferred_element_type=jnp.float32) + b2_ref[...]
    if not last:
        o = jnp.maximum(o, 0.0)
    out_ref[...] = o


_BE = 2000
_RB = 400


def kernel(x, edge_index, edge_attr, batch, node_table, We, be, eps,
           W1, b1, g1, bt1, m1, v1, W2, b2, go, bo, mo, vo):
    f32 = jnp.float32
    src2 = edge_index[0].reshape(E // 128, 128)
    dst2 = edge_index[1].reshape(E // 128, 128)
    h0row = node_table[0]

    s1 = g1 / jnp.sqrt(v1 + 1e-5)
    W1f = W1 * s1[:, None, :]
    b1f = b1 * s1 + (bt1 - m1 * s1)
    s2 = go / jnp.sqrt(vo + 1e-5)
    W2f = W2 * s2[:, None, :]
    b2f = b2 * s2 + (bo - mo * s2)

    bee = be.at[0].add(h0row).reshape(L, 1, D)

    def _enc(ls):
        nl = len(ls)
        return pl.pallas_call(
            _enc_body,
            grid=(E // _BE,),
            in_specs=[
                pl.BlockSpec((_BE, 7), lambda i: (i, 0)),
                pl.BlockSpec((nl, 7, D), lambda i: (0, 0, 0)),
                pl.BlockSpec((nl, 1, D), lambda i: (0, 0, 0)),
            ],
            out_specs=[pl.BlockSpec((_BE // 2, D), lambda i: (i, 0)) for _ in ls],
            out_shape=[jax.ShapeDtypeStruct((E // 2, D), jnp.int32) for _ in ls],
        )(edge_attr, We[ls[0]:ls[-1] + 1], bee[ls[0]:ls[-1] + 1])

    sc_first = _make_sc(False)
    sc_rest = _make_sc(True)

    e0 = _enc([0])[0]
    h = jnp.broadcast_to(node_table[0:1], (N, D))
    agg_first = sc_first(e0, src2, dst2, h)
    e12 = _enc([1, 2])
    e_all = [e0, e12[0], e12[1]]
    for l in range(L):
        agg2 = agg_first if l == 0 else sc_rest(e_all[l], src2, dst2, h)
        epv = jnp.full((1, D), 1.0 + eps[l], f32)
        h = pl.pallas_call(
            functools.partial(_mlp_body, l == L - 1),
            grid=(N // _RB,),
            in_specs=[
                pl.BlockSpec((_RB, D), lambda i: (i, 0)),
                pl.BlockSpec((NC, _RB, D), lambda i: (0, i, 0)),
                pl.BlockSpec((1, D), lambda i: (0, 0)),
                pl.BlockSpec((D, H), lambda i: (0, 0)),
                pl.BlockSpec((1, H), lambda i: (0, 0)),
                pl.BlockSpec((H, D), lambda i: (0, 0)),
                pl.BlockSpec((1, D), lambda i: (0, 0)),
            ],
            out_specs=pl.BlockSpec((_RB, D), lambda i: (i, 0)),
            out_shape=jax.ShapeDtypeStruct((N, D), f32),
        )(h, agg2, epv, W1f[l], b1f[l].reshape(1, H), W2f[l], b2f[l].reshape(1, D))
    return h


# speedup vs baseline: 1.6270x; 1.6270x over previous
import functools

import jax
import jax.numpy as jnp
from jax import lax
from jax.experimental import pallas as pl
from jax.experimental.pallas import tpu as pltpu
from jax.experimental.pallas import tpu_sc as plsc

N = 10000
E = 320000
D = 128
H = 256
L = 3

NC = 2
NS = 16
NWORK = NC * NS
CHUNK = 128
NCHUNKS = E // CHUNK
ITERS = (NCHUNKS + NWORK - 1) // NWORK
OUTER = (ITERS + 1) // 2
RPS = 624


def _sc_body(do_gather, e_hbm, src_hbm, dst_hbm, h_hbm, out_hbm,
             src0, src1, dst0, dst1, ev0, ev1, agg_sh,
             sl0, sl1, sd0, sd1, se0, se1, sg0, sg1):
    c = lax.axis_index("c")
    s = lax.axis_index("s")
    wid = s * NC + c
    sets = ((src0, dst0, ev0, sl0, sd0, se0, sg0),
            (src1, dst1, ev1, sl1, sd1, se1, sg1))

    def _zrow(r, carry):
        for j in range(8):
            ev0[r, pl.ds(j * 16, 16)] = jnp.zeros((16,), jnp.float32)
        return carry
    lax.fori_loop(0, CHUNK, _zrow, 0)
    for off, n in ((0, 128), (128, 128), (256, 128), (384, 128), (512, 112)):
        pltpu.sync_copy(ev0.at[pl.ds(0, n)],
                        agg_sh.at[pl.ds(s * RPS + off, n)])

    @pl.when(s == NS - 1)
    def _ztail():
        pltpu.sync_copy(ev0.at[pl.ds(0, 16)], agg_sh.at[pl.ds(NS * RPS, 16)])
    plsc.subcore_barrier()

    def _lin(cid, bufs, issue):
        srcb, dstb, evb = bufs[0], bufs[1], bufs[2]
        slb, sdb, seb = bufs[3], bufs[4], bufs[5]
        for hbm, vb, sem in ((src_hbm, srcb, slb), (dst_hbm, dstb, sdb)):
            cp = pltpu.make_async_copy(hbm.at[pl.ds(cid, 1)], vb, sem)
            cp.start() if issue else cp.wait()
        cp = pltpu.make_async_copy(e_hbm.at[pl.ds(cid * CHUNK, CHUNK)], evb, seb)
        cp.start() if issue else cp.wait()

    def _gat(bufs, issue):
        if issue:
            pltpu.async_copy(h_hbm.at[bufs[0].at[0]], bufs[2], bufs[6], add=True)
        else:
            pltpu.make_async_copy(h_hbm.at[bufs[0].at[0]], bufs[2], bufs[6]).wait()

    _lin(wid, sets[0], True)
    _lin(wid + NWORK, sets[1], True)
    if do_gather:
        _lin(wid, sets[0], False)
        _gat(sets[0], True)

    def _outer(o, carry):
        for b in range(2):
            cid = wid + (2 * o + b) * NWORK
            bufs = sets[b]
            nbufs = sets[1 - b]
            dstb, evb = bufs[1], bufs[2]

            @pl.when(cid < NCHUNKS)
            def _():
                if do_gather:
                    @pl.when(cid + NWORK < NCHUNKS)
                    def _():
                        _lin(cid + NWORK, nbufs, False)
                        _gat(nbufs, True)
                    _gat(bufs, False)
                else:
                    _lin(cid, bufs, False)

                def _row(r, carry2):
                    for j in range(8):
                        sl = pl.ds(j * 16, 16)
                        evb[r, sl] = jnp.maximum(evb[r, sl], 0.0)
                    return carry2
                lax.fori_loop(0, CHUNK, _row, 0)

                pltpu.sync_copy(evb, agg_sh.at[dstb.at[0]], add=True)

                @pl.when(cid + 2 * NWORK < NCHUNKS)
                def _():
                    _lin(cid + 2 * NWORK, bufs, True)
        return carry
    lax.fori_loop(0, OUTER, _outer, 0)
    plsc.subcore_barrier()

    for off, n in ((0, 128), (128, 128), (256, 128), (384, 128), (512, 112)):
        pltpu.sync_copy(agg_sh.at[pl.ds(s * RPS + off, n)],
                        out_hbm.at[c, pl.ds(s * RPS + off, n)])

    @pl.when(s == NS - 1)
    def _wtail():
        pltpu.sync_copy(agg_sh.at[pl.ds(NS * RPS, 16)],
                        out_hbm.at[c, pl.ds(NS * RPS, 16)])


def _make_sc(do_gather):
    mesh = plsc.VectorSubcoreMesh(core_axis_name="c", subcore_axis_name="s")
    return pl.kernel(
        functools.partial(_sc_body, do_gather),
        out_type=jax.ShapeDtypeStruct((NC, N, D), jnp.float32),
        mesh=mesh,
        scratch_types=[
            pltpu.VMEM((1, 128), jnp.int32),
            pltpu.VMEM((1, 128), jnp.int32),
            pltpu.VMEM((1, 128), jnp.int32),
            pltpu.VMEM((1, 128), jnp.int32),
            pltpu.VMEM((CHUNK, D), jnp.float32),
            pltpu.VMEM((CHUNK, D), jnp.float32),
            pltpu.VMEM_SHARED((N, D), jnp.float32),
        ] + [pltpu.SemaphoreType.DMA] * 8,
    )


def _enc_body(ea_ref, we_ref, be_ref, *out_refs):
    ea = ea_ref[...]
    for l, out_ref in enumerate(out_refs):
        out_ref[...] = (jnp.dot(ea, we_ref[l], preferred_element_type=jnp.float32)
                        + be_ref[l])


def _mlp_body(last, h_ref, agg_ref, ep_ref, w1_ref, b1_ref, w2_ref, b2_ref, out_ref):
    t = ep_ref[...] * h_ref[...] + agg_ref[0] + agg_ref[1]
    t = jnp.maximum(jnp.dot(t, w1_ref[...], preferred_element_type=jnp.float32)
                    + b1_ref[...], 0.0)
    o = jnp.dot(t, w2_ref[...], preferred_element_type=jnp.float32) + b2_ref[...]
    if not last:
        o = jnp.maximum(o, 0.0)
    out_ref[...] = o


_BE = 2000
_RB = 400


def kernel(x, edge_index, edge_attr, batch, node_table, We, be, eps,
           W1, b1, g1, bt1, m1, v1, W2, b2, go, bo, mo, vo):
    f32 = jnp.float32
    src2 = edge_index[0].reshape(E // 128, 128)
    dst2 = edge_index[1].reshape(E // 128, 128)
    h0row = node_table[0]

    s1 = g1 / jnp.sqrt(v1 + 1e-5)
    W1f = W1 * s1[:, None, :]
    b1f = b1 * s1 + (bt1 - m1 * s1)
    s2 = go / jnp.sqrt(vo + 1e-5)
    W2f = W2 * s2[:, None, :]
    b2f = b2 * s2 + (bo - mo * s2)

    bee = be.at[0].add(h0row).reshape(L, 1, D)

    def _enc(ls):
        nl = len(ls)
        return pl.pallas_call(
            _enc_body,
            grid=(E // _BE,),
            in_specs=[
                pl.BlockSpec((_BE, 7), lambda i: (i, 0)),
                pl.BlockSpec((nl, 7, D), lambda i: (0, 0, 0)),
                pl.BlockSpec((nl, 1, D), lambda i: (0, 0, 0)),
            ],
            out_specs=[pl.BlockSpec((_BE, D), lambda i: (i, 0)) for _ in ls],
            out_shape=[jax.ShapeDtypeStruct((E, D), f32) for _ in ls],
        )(edge_attr, We[ls[0]:ls[-1] + 1], bee[ls[0]:ls[-1] + 1])

    sc_first = _make_sc(False)
    sc_rest = _make_sc(True)

    def _mlp(l, h, agg2):
        epv = jnp.full((1, D), 1.0 + eps[l], f32)
        return pl.pallas_call(
            functools.partial(_mlp_body, l == L - 1),
            grid=(N // _RB,),
            in_specs=[
                pl.BlockSpec((_RB, D), lambda i: (i, 0)),
                pl.BlockSpec((NC, _RB, D), lambda i: (0, i, 0)),
                pl.BlockSpec((1, D), lambda i: (0, 0)),
                pl.BlockSpec((D, H), lambda i: (0, 0)),
                pl.BlockSpec((1, H), lambda i: (0, 0)),
                pl.BlockSpec((H, D), lambda i: (0, 0)),
                pl.BlockSpec((1, D), lambda i: (0, 0)),
            ],
            out_specs=pl.BlockSpec((_RB, D), lambda i: (i, 0)),
            out_shape=jax.ShapeDtypeStruct((N, D), f32),
        )(h, agg2, epv, W1f[l], b1f[l].reshape(1, H), W2f[l], b2f[l].reshape(1, D))

    h = jnp.broadcast_to(node_table[0:1], (N, D))
    e0 = _enc([0])[0]
    agg = sc_first(e0, src2, dst2, h)
    e1 = _enc([1])[0]
    h = _mlp(0, h, agg)
    agg = sc_rest(e1, src2, dst2, h)
    e2 = _enc([2])[0]
    h = _mlp(1, h, agg)
    agg = sc_rest(e2, src2, dst2, h)
    return _mlp(2, h, agg)


# speedup vs baseline: 1.6633x; 1.0223x over previous
import functools

import jax
import jax.numpy as jnp
from jax import lax
from jax.experimental import pallas as pl
from jax.experimental.pallas import tpu as pltpu
from jax.experimental.pallas import tpu_sc as plsc

N = 10000
E = 320000
D = 128
H = 256
L = 3

NC = 2
NS = 16
NWORK = NC * NS
CHUNK = 128
NCHUNKS = E // CHUNK
ITERS = (NCHUNKS + NWORK - 1) // NWORK
OUTER = (ITERS + 1) // 2
RPS = 624


def _sc_body(do_gather, e_hbm, src_hbm, dst_hbm, h_hbm, out_hbm,
             src0, src1, src2b, dst0, dst1, dst2b, ev0, ev1, ev2, agg_sh,
             *sems):
    c = lax.axis_index("c")
    s = lax.axis_index("s")
    wid = s * NC + c
    sets = tuple((srcb, dstb, evb) + tuple(sems[5 * i:5 * i + 5])
                 for i, (srcb, dstb, evb) in enumerate(
                     ((src0, dst0, ev0), (src1, dst1, ev1), (src2b, dst2b, ev2))))

    def _zrow(r, carry):
        for j in range(8):
            ev0[r, pl.ds(j * 16, 16)] = jnp.zeros((16,), jnp.float32)
        return carry
    lax.fori_loop(0, CHUNK, _zrow, 0)
    for off, n in ((0, 128), (128, 128), (256, 128), (384, 128), (512, 112)):
        pltpu.sync_copy(ev0.at[pl.ds(0, n)],
                        agg_sh.at[pl.ds(s * RPS + off, n)])

    @pl.when(s == NS - 1)
    def _ztail():
        pltpu.sync_copy(ev0.at[pl.ds(0, 16)], agg_sh.at[pl.ds(NS * RPS, 16)])
    plsc.subcore_barrier()

    def _lin(cid, bufs, issue):
        srcb, dstb, evb, slb, sdb, seb = bufs[:6]
        for hbm, vb, sem in ((src_hbm, srcb, slb), (dst_hbm, dstb, sdb)):
            cp = pltpu.make_async_copy(hbm.at[pl.ds(cid, 1)], vb, sem)
            cp.start() if issue else cp.wait()
        cp = pltpu.make_async_copy(e_hbm.at[pl.ds(cid * CHUNK, CHUNK)], evb, seb)
        cp.start() if issue else cp.wait()

    def _gat(bufs, issue):
        if issue:
            pltpu.async_copy(h_hbm.at[bufs[0].at[0]], bufs[2], bufs[6], add=True)
        else:
            pltpu.make_async_copy(h_hbm.at[bufs[0].at[0]], bufs[2], bufs[6]).wait()

    def _sca(bufs, issue):
        if issue:
            pltpu.async_copy(bufs[2], agg_sh.at[bufs[1].at[0]], bufs[7], add=True)
        else:
            pltpu.make_async_copy(bufs[2], agg_sh.at[bufs[1].at[0]],
                                  bufs[7]).wait()

    _lin(wid, sets[0], True)
    _lin(wid + NWORK, sets[1], True)
    _lin(wid + 2 * NWORK, sets[2], True)
    if do_gather:
        _lin(wid, sets[0], False)
        _gat(sets[0], True)

    def _outer(o, carry):
        for b in range(3):
            t = 3 * o + b
            cid = wid + t * NWORK
            st = sets[b]
            st1 = sets[(b + 1) % 3]
            st2 = sets[(b + 2) % 3]
            evb = st[2]

            @pl.when(cid < NCHUNKS)
            def _():
                if do_gather:
                    @pl.when(cid + NWORK < NCHUNKS)
                    def _():
                        _lin(cid + NWORK, st1, False)
                        _gat(st1, True)
                    _gat(st, False)
                else:
                    _lin(cid, st, False)

                def _row(r, carry2):
                    for j in range(8):
                        sl = pl.ds(j * 16, 16)
                        evb[r, sl] = jnp.maximum(evb[r, sl], 0.0)
                    return carry2
                lax.fori_loop(0, CHUNK, _row, 0)

                _sca(st, True)

                @pl.when(t >= 1)
                def _():
                    _sca(st2, False)

                    @pl.when(cid + 2 * NWORK < NCHUNKS)
                    def _():
                        _lin(cid + 2 * NWORK, st2, True)
        return carry
    lax.fori_loop(0, (ITERS + 2) // 3, _outer, 0)

    last_full = NCHUNKS - (ITERS - 1) * NWORK

    @pl.when(wid < last_full)
    def _():
        _sca(sets[(ITERS - 1) % 3], False)

    @pl.when(wid >= last_full)
    def _():
        _sca(sets[(ITERS - 2) % 3], False)
    plsc.subcore_barrier()

    for off, n in ((0, 128), (128, 128), (256, 128), (384, 128), (512, 112)):
        pltpu.sync_copy(agg_sh.at[pl.ds(s * RPS + off, n)],
                        out_hbm.at[c, pl.ds(s * RPS + off, n)])

    @pl.when(s == NS - 1)
    def _wtail():
        pltpu.sync_copy(agg_sh.at[pl.ds(NS * RPS, 16)],
                        out_hbm.at[c, pl.ds(NS * RPS, 16)])


def _make_sc(do_gather):
    mesh = plsc.VectorSubcoreMesh(core_axis_name="c", subcore_axis_name="s")
    return pl.kernel(
        functools.partial(_sc_body, do_gather),
        out_type=jax.ShapeDtypeStruct((NC, N, D), jnp.float32),
        mesh=mesh,
        scratch_types=[
            pltpu.VMEM((1, 128), jnp.int32),
            pltpu.VMEM((1, 128), jnp.int32),
            pltpu.VMEM((1, 128), jnp.int32),
            pltpu.VMEM((1, 128), jnp.int32),
            pltpu.VMEM((1, 128), jnp.int32),
            pltpu.VMEM((1, 128), jnp.int32),
            pltpu.VMEM((CHUNK, D), jnp.float32),
            pltpu.VMEM((CHUNK, D), jnp.float32),
            pltpu.VMEM((CHUNK, D), jnp.float32),
            pltpu.VMEM_SHARED((N, D), jnp.float32),
        ] + [pltpu.SemaphoreType.DMA] * 15,
    )


def _enc_body(ea_ref, we_ref, be_ref, *out_refs):
    ea = ea_ref[...]
    for l, out_ref in enumerate(out_refs):
        out_ref[...] = (jnp.dot(ea, we_ref[l], preferred_element_type=jnp.float32)
                        + be_ref[l])


def _mlp_body(last, h_ref, agg_ref, ep_ref, w1_ref, b1_ref, w2_ref, b2_ref, out_ref):
    t = ep_ref[...] * h_ref[...] + agg_ref[0] + agg_ref[1]
    t = jnp.maximum(jnp.dot(t, w1_ref[...], preferred_element_type=jnp.float32)
                    + b1_ref[...], 0.0)
    o = jnp.dot(t, w2_ref[...], preferred_element_type=jnp.float32) + b2_ref[...]
    if not last:
        o = jnp.maximum(o, 0.0)
    out_ref[...] = o


_BE = 2000
_RB = 400


def kernel(x, edge_index, edge_attr, batch, node_table, We, be, eps,
           W1, b1, g1, bt1, m1, v1, W2, b2, go, bo, mo, vo):
    f32 = jnp.float32
    src2 = edge_index[0].reshape(E // 128, 128)
    dst2 = edge_index[1].reshape(E // 128, 128)
    h0row = node_table[0]

    s1 = g1 / jnp.sqrt(v1 + 1e-5)
    W1f = W1 * s1[:, None, :]
    b1f = b1 * s1 + (bt1 - m1 * s1)
    s2 = go / jnp.sqrt(vo + 1e-5)
    W2f = W2 * s2[:, None, :]
    b2f = b2 * s2 + (bo - mo * s2)

    bee = be.at[0].add(h0row).reshape(L, 1, D)

    def _enc(ls):
        nl = len(ls)
        return pl.pallas_call(
            _enc_body,
            grid=(E // _BE,),
            in_specs=[
                pl.BlockSpec((_BE, 7), lambda i: (i, 0)),
                pl.BlockSpec((nl, 7, D), lambda i: (0, 0, 0)),
                pl.BlockSpec((nl, 1, D), lambda i: (0, 0, 0)),
            ],
            out_specs=[pl.BlockSpec((_BE, D), lambda i: (i, 0)) for _ in ls],
            out_shape=[jax.ShapeDtypeStruct((E, D), f32) for _ in ls],
        )(edge_attr, We[ls[0]:ls[-1] + 1], bee[ls[0]:ls[-1] + 1])

    sc_first = _make_sc(False)
    sc_rest = _make_sc(True)

    def _mlp(l, h, agg2):
        epv = jnp.full((1, D), 1.0 + eps[l], f32)
        return pl.pallas_call(
            functools.partial(_mlp_body, l == L - 1),
            grid=(N // _RB,),
            in_specs=[
                pl.BlockSpec((_RB, D), lambda i: (i, 0)),
                pl.BlockSpec((NC, _RB, D), lambda i: (0, i, 0)),
                pl.BlockSpec((1, D), lambda i: (0, 0)),
                pl.BlockSpec((D, H), lambda i: (0, 0)),
                pl.BlockSpec((1, H), lambda i: (0, 0)),
                pl.BlockSpec((H, D), lambda i: (0, 0)),
                pl.BlockSpec((1, D), lambda i: (0, 0)),
            ],
            out_specs=pl.BlockSpec((_RB, D), lambda i: (i, 0)),
            out_shape=jax.ShapeDtypeStruct((N, D), f32),
        )(h, agg2, epv, W1f[l], b1f[l].reshape(1, H), W2f[l], b2f[l].reshape(1, D))

    h = jnp.broadcast_to(node_table[0:1], (N, D))
    e0 = _enc([0])[0]
    agg = sc_first(e0, src2, dst2, h)
    e1 = _enc([1])[0]
    h = _mlp(0, h, agg)
    agg = sc_rest(e1, src2, dst2, h)
    e2 = _enc([2])[0]
    h = _mlp(1, h, agg)
    agg = sc_rest(e2, src2, dst2, h)
    return _mlp(2, h, agg)


# speedup vs baseline: 2.1140x; 1.2709x over previous
import functools

import jax
import jax.numpy as jnp
from jax import lax
from jax.experimental import pallas as pl
from jax.experimental.pallas import tpu as pltpu
from jax.experimental.pallas import tpu_sc as plsc

N = 10000
E = 320000
D = 128
H = 256
L = 3

NC = 2
NS = 16
NWORK = NC * NS
CHUNK = 128
NCHUNKS = E // CHUNK
ITERS = (NCHUNKS + NWORK - 1) // NWORK
OUTER = (ITERS + 1) // 2
RPS = 624


def _sc_body(do_gather, e_hbm, src_hbm, dst_hbm, h_hbm, out_hbm,
             src0, src1, src2b, dst0, dst1, dst2b, ev0, ev1, ev2, agg_sh,
             *sems):
    c = lax.axis_index("c")
    s = lax.axis_index("s")
    wid = s * NC + c
    sets = tuple((srcb, dstb, evb) + tuple(sems[5 * i:5 * i + 5])
                 for i, (srcb, dstb, evb) in enumerate(
                     ((src0, dst0, ev0), (src1, dst1, ev1), (src2b, dst2b, ev2))))

    def _zrow(r, carry):
        for j in range(8):
            ev0[r, pl.ds(j * 16, 16)] = jnp.zeros((16,), jnp.float32)
        return carry
    lax.fori_loop(0, CHUNK, _zrow, 0)
    for off, n in ((0, 128), (128, 128), (256, 128), (384, 128), (512, 112)):
        pltpu.sync_copy(ev0.at[pl.ds(0, n)],
                        agg_sh.at[pl.ds(s * RPS + off, n)])

    @pl.when(s == NS - 1)
    def _ztail():
        pltpu.sync_copy(ev0.at[pl.ds(0, 16)], agg_sh.at[pl.ds(NS * RPS, 16)])
    plsc.subcore_barrier()

    def _lin(cid, bufs, issue):
        srcb, dstb, evb, slb, sdb, seb = bufs[:6]
        for hbm, vb, sem in ((src_hbm, srcb, slb), (dst_hbm, dstb, sdb)):
            cp = pltpu.make_async_copy(hbm.at[pl.ds(cid, 1)], vb, sem)
            cp.start() if issue else cp.wait()
        cp = pltpu.make_async_copy(e_hbm.at[pl.ds(cid * CHUNK, CHUNK)], evb, seb)
        cp.start() if issue else cp.wait()

    def _gat(bufs, issue):
        if issue:
            pltpu.async_copy(h_hbm.at[bufs[0].at[0]], bufs[2], bufs[6], add=True)
        else:
            pltpu.make_async_copy(h_hbm.at[bufs[0].at[0]], bufs[2], bufs[6]).wait()

    def _sca(bufs, issue):
        if issue:
            pltpu.async_copy(bufs[2], agg_sh.at[bufs[1].at[0]], bufs[7], add=True)
        else:
            pltpu.make_async_copy(bufs[2], agg_sh.at[bufs[1].at[0]],
                                  bufs[7]).wait()

    _lin(wid, sets[0], True)
    _lin(wid + NWORK, sets[1], True)
    _lin(wid + 2 * NWORK, sets[2], True)
    if do_gather:
        _lin(wid, sets[0], False)
        _gat(sets[0], True)

    def _outer(o, carry):
        for b in range(3):
            t = 3 * o + b
            cid = wid + t * NWORK
            st = sets[b]
            st1 = sets[(b + 1) % 3]
            st2 = sets[(b + 2) % 3]
            evb = st[2]

            @pl.when(cid < NCHUNKS)
            def _():
                if do_gather:
                    @pl.when(cid + NWORK < NCHUNKS)
                    def _():
                        _lin(cid + NWORK, st1, False)
                        _gat(st1, True)
                    _gat(st, False)
                else:
                    _lin(cid, st, False)

                def _row(r, carry2):
                    for j in range(8):
                        sl = pl.ds(j * 16, 16)
                        evb[r, sl] = jnp.maximum(evb[r, sl], 0.0)
                    return carry2
                lax.fori_loop(0, CHUNK, _row, 0)

                _sca(st, True)

                @pl.when(t >= 1)
                def _():
                    _sca(st2, False)

                    @pl.when(cid + 2 * NWORK < NCHUNKS)
                    def _():
                        _lin(cid + 2 * NWORK, st2, True)
        return carry
    lax.fori_loop(0, (ITERS + 2) // 3, _outer, 0)

    last_full = NCHUNKS - (ITERS - 1) * NWORK

    @pl.when(wid < last_full)
    def _():
        _sca(sets[(ITERS - 1) % 3], False)

    @pl.when(wid >= last_full)
    def _():
        _sca(sets[(ITERS - 2) % 3], False)
    plsc.subcore_barrier()

    for off, n in ((0, 128), (128, 128), (256, 128), (384, 128), (512, 112)):
        pltpu.sync_copy(agg_sh.at[pl.ds(s * RPS + off, n)],
                        out_hbm.at[c, pl.ds(s * RPS + off, n)])

    @pl.when(s == NS - 1)
    def _wtail():
        pltpu.sync_copy(agg_sh.at[pl.ds(NS * RPS, 16)],
                        out_hbm.at[c, pl.ds(NS * RPS, 16)])


def _make_sc(do_gather):
    mesh = plsc.VectorSubcoreMesh(core_axis_name="c", subcore_axis_name="s")
    return pl.kernel(
        functools.partial(_sc_body, do_gather),
        out_type=jax.ShapeDtypeStruct((NC, N, D), jnp.float32),
        mesh=mesh,
        scratch_types=[
            pltpu.VMEM((1, 128), jnp.int32),
            pltpu.VMEM((1, 128), jnp.int32),
            pltpu.VMEM((1, 128), jnp.int32),
            pltpu.VMEM((1, 128), jnp.int32),
            pltpu.VMEM((1, 128), jnp.int32),
            pltpu.VMEM((1, 128), jnp.int32),
            pltpu.VMEM((CHUNK, D), jnp.float32),
            pltpu.VMEM((CHUNK, D), jnp.float32),
            pltpu.VMEM((CHUNK, D), jnp.float32),
            pltpu.VMEM_SHARED((N, D), jnp.float32),
        ] + [pltpu.SemaphoreType.DMA] * 15,
    )


def _enc_body(ea_ref, we_ref, be_ref, *out_refs):
    ea = ea_ref[...]
    for l, out_ref in enumerate(out_refs):
        out_ref[...] = (lax.dot_general(
            ea, we_ref[l], (((0,), (0,)), ((), ())),
            preferred_element_type=jnp.float32) + be_ref[l])


def _mlp_body(last, h_ref, agg_ref, ep_ref, w1_ref, b1_ref, w2_ref, b2_ref, out_ref):
    t = ep_ref[...] * h_ref[...] + agg_ref[0] + agg_ref[1]
    t = jnp.maximum(jnp.dot(t, w1_ref[...], preferred_element_type=jnp.float32)
                    + b1_ref[...], 0.0)
    o = jnp.dot(t, w2_ref[...], preferred_element_type=jnp.float32) + b2_ref[...]
    if not last:
        o = jnp.maximum(o, 0.0)
    out_ref[...] = o


_BE = 2560
_RB = 400


def kernel(x, edge_index, edge_attr, batch, node_table, We, be, eps,
           W1, b1, g1, bt1, m1, v1, W2, b2, go, bo, mo, vo):
    f32 = jnp.float32
    src2 = edge_index[0].reshape(E // 128, 128)
    dst2 = edge_index[1].reshape(E // 128, 128)
    h0row = node_table[0]

    s1 = g1 / jnp.sqrt(v1 + 1e-5)
    W1f = W1 * s1[:, None, :]
    b1f = b1 * s1 + (bt1 - m1 * s1)
    s2 = go / jnp.sqrt(vo + 1e-5)
    W2f = W2 * s2[:, None, :]
    b2f = b2 * s2 + (bo - mo * s2)

    bee = be.at[0].add(h0row).reshape(L, 1, D)
    eaT = edge_attr.T

    def _enc(ls):
        nl = len(ls)
        return pl.pallas_call(
            _enc_body,
            grid=(E // _BE,),
            in_specs=[
                pl.BlockSpec((7, _BE), lambda i: (0, i)),
                pl.BlockSpec((nl, 7, D), lambda i: (0, 0, 0)),
                pl.BlockSpec((nl, 1, D), lambda i: (0, 0, 0)),
            ],
            out_specs=[pl.BlockSpec((_BE, D), lambda i: (i, 0)) for _ in ls],
            out_shape=[jax.ShapeDtypeStruct((E, D), f32) for _ in ls],
        )(eaT, We[ls[0]:ls[-1] + 1], bee[ls[0]:ls[-1] + 1])

    sc_first = _make_sc(False)
    sc_rest = _make_sc(True)

    def _mlp(l, h, agg2):
        epv = jnp.full((1, D), 1.0 + eps[l], f32)
        return pl.pallas_call(
            functools.partial(_mlp_body, l == L - 1),
            grid=(N // _RB,),
            in_specs=[
                pl.BlockSpec((_RB, D), lambda i: (i, 0)),
                pl.BlockSpec((NC, _RB, D), lambda i: (0, i, 0)),
                pl.BlockSpec((1, D), lambda i: (0, 0)),
                pl.BlockSpec((D, H), lambda i: (0, 0)),
                pl.BlockSpec((1, H), lambda i: (0, 0)),
                pl.BlockSpec((H, D), lambda i: (0, 0)),
                pl.BlockSpec((1, D), lambda i: (0, 0)),
            ],
            out_specs=pl.BlockSpec((_RB, D), lambda i: (i, 0)),
            out_shape=jax.ShapeDtypeStruct((N, D), f32),
        )(h, agg2, epv, W1f[l], b1f[l].reshape(1, H), W2f[l], b2f[l].reshape(1, D))

    h = jnp.broadcast_to(node_table[0:1], (N, D))
    e0 = _enc([0])[0]
    agg = sc_first(e0, src2, dst2, h)
    e1 = _enc([1])[0]
    h = _mlp(0, h, agg)
    agg = sc_rest(e1, src2, dst2, h)
    e2 = _enc([2])[0]
    h = _mlp(1, h, agg)
    agg = sc_rest(e2, src2, dst2, h)
    return _mlp(2, h, agg)
